# Initial kernel scaffold; baseline (speedup 1.0000x reference)
import jax
import jax.numpy as jnp
from jax.experimental import pallas as pl


def kernel(pred_boxes, gt_boxes, mask_gt, pred_scores, gt_labels):
    raise NotImplementedError("write your pallas kernel here")


# speedup vs baseline: 12.3107x; 12.3107x over previous
import math

import jax
import jax.numpy as jnp
from jax import lax
from jax.experimental import pallas as pl
from jax.experimental.pallas import tpu as pltpu

_NC = 80
_TOPK = 10
_EPS = 1e-09
_DIS = 2.5
_AT = 2000
_NSLOT = 16


def _fiota(shape, dim):
    return lax.broadcasted_iota(jnp.int32, shape, dim).astype(jnp.float32)


def _sum_classes_like_xla(e):
    z = e[:, 0:8]
    for j in range(1, _NC // 8):
        z = z + e[:, 8 * j : 8 * j + 8]
    a = z[:, 0:4] + z[:, 4:8]
    b = a[:, 0:2] + a[:, 2:4]
    return b[:, 0:1] + b[:, 1:2]


def _sum10_like_xla(vals):
    z = [vals[i] + vals[8 + i] if 8 + i < len(vals) else vals[i] for i in range(8)]
    a = [z[i] + z[i + 4] for i in range(4)]
    b = [a[0] + a[2], a[1] + a[3]]
    return b[0] + b[1]


def _pack_slots(cols, pad):
    G = cols[0].shape[0]
    slot = _fiota((G, _NSLOT), 1)
    acc = jnp.full((G, _NSLOT), pad, jnp.float32)
    for j, c in enumerate(cols):
        acc = jnp.where(slot == float(j), c, acc)
    return acc


def _body(pbt_ref, g_ref, ps_ref, out_ref, cval_ref, cidx_ref, ival_ref):
    t = pl.program_id(1)
    nt = pl.num_programs(1)
    G = g_ref.shape[1]
    A = out_ref.shape[2]

    x1 = pbt_ref[0, 0, 0:1, :]
    y1 = pbt_ref[0, 0, 1:2, :]
    x2 = pbt_ref[0, 0, 2:3, :]
    y2 = pbt_ref[0, 0, 3:4, :]
    t1 = pbt_ref[0, 0, 4:5, :]
    gx1 = g_ref[0, :, 0:1]
    gy1 = g_ref[0, :, 1:2]
    gx2 = g_ref[0, :, 2:3]
    gy2 = g_ref[0, :, 3:4]
    t2 = g_ref[0, :, 4:5]
    mg = g_ref[0, :, 5:6]
    glf = g_ref[0, :, 6:7]

    cx = (x1 + x2) / 2.0
    cy = (y1 + y2) / 2.0
    in_boxes = (cx - gx1 > 0.0) & (cy - gy1 > 0.0) & (gx2 - cx > 0.0) & (gy2 - cy > 0.0)
    gcx = (gx1 + gx2) / 2.0
    gcy = (gy1 + gy2) / 2.0
    b2x1 = gcx - _DIS
    b2y1 = gcy - _DIS
    b2x2 = gcx + _DIS
    b2y2 = gcy + _DIS
    in_centers = (cx - b2x1 > 0.0) & (cy - b2y1 > 0.0) & (b2x2 - cx > 0.0) & (b2y2 - cy > 0.0)
    both = in_boxes & in_centers

    w1 = x2 - x1
    h1 = y2 - y1 + _EPS
    w2 = gx2 - gx1
    h2 = gy2 - gy1 + _EPS
    iw = jnp.maximum(jnp.minimum(x2, gx2) - jnp.maximum(x1, gx1), 0.0)
    ih = jnp.maximum(jnp.minimum(y2, gy2) - jnp.maximum(y1, gy1), 0.0)
    inter = iw * ih
    union = w1 * h1 + w2 * h2 - inter + _EPS
    iou = inter / union
    cw = jnp.maximum(x2, gx2) - jnp.minimum(x1, gx1)
    ch = jnp.maximum(y2, gy2) - jnp.minimum(y1, gy1)
    c2 = cw * cw + ch * ch + _EPS
    dx = gx1 + gx2 - x1 - x2
    dy = gy1 + gy2 - y1 - y2
    d2 = (dx * dx + dy * dy) / 4.0
    tdiff = t2 - t1
    v = 4.0 / math.pi**2 * (tdiff * tdiff)
    alpha = v / (v - iou + (1.0 + _EPS))
    ciou = iou - (d2 / c2 + v * alpha)

    p = jnp.clip(ps_ref[0], 1e-07, 1.0 - 1e-07)
    logp = jnp.log(p)
    log1mp = jnp.log1p(-p)
    s_col = _sum_classes_like_xla(-log1mp)
    onehot = jnp.where(_fiota((G, _NC), 1) == glf, 1.0, 0.0)
    dims = (((1,), (1,)), ((), ()))
    hi = jax.lax.Precision.HIGHEST
    lp_g = lax.dot_general(onehot, logp, dims, precision=hi, preferred_element_type=jnp.float32)
    l1_g = lax.dot_general(onehot, log1mp, dims, precision=hi, preferred_element_type=jnp.float32)
    ones_g = jnp.ones((G, 1), jnp.float32)
    s_row = lax.dot_general(ones_g, s_col, dims, precision=hi, preferred_element_type=jnp.float32)
    bce = (s_row - lp_g + l1_g) / float(_NC)

    cost = bce + 3.0 * ciou + 100000.0 * (1.0 - jnp.where(both, 1.0, 0.0))
    cost = cost + 1e9 * (1.0 - mg)
    iou_pos = jnp.clip(iou, 0.0, 1.0)

    lane = _fiota((G, _AT), 1)
    big = jnp.float32(1e9)
    inf = jnp.float32(jnp.inf)
    base = (t * _AT).astype(jnp.float32)

    cvals, cidxs = [], []
    work = cost
    for _ in range(_TOPK):
        mv = jnp.min(work, axis=1, keepdims=True)
        li = jnp.min(jnp.where(work == mv, lane, big), axis=1, keepdims=True)
        work = jnp.where(lane == li, inf, work)
        cvals.append(mv)
        cidxs.append(li + base)
    cval_ref[t] = _pack_slots(cvals, jnp.inf)
    cidx_ref[t] = _pack_slots(cidxs, 1e9)

    ivals = []
    work = iou_pos
    for _ in range(_TOPK):
        mv = jnp.max(work, axis=1, keepdims=True)
        li = jnp.min(jnp.where(work == mv, lane, big), axis=1, keepdims=True)
        work = jnp.where(lane == li, -1.0, work)
        ivals.append(mv)
    ival_ref[t] = _pack_slots(ivals, -1.0)

    @pl.when(t == nt - 1)
    def _finalize():
        av = jnp.concatenate([cval_ref[tt] for tt in range(nt)], axis=1)
        ai = jnp.concatenate([cidx_ref[tt] for tt in range(nt)], axis=1)
        iv = jnp.concatenate([ival_ref[tt] for tt in range(nt)], axis=1)
        nslots = nt * _NSLOT
        slot = _fiota((G, nslots), 1)

        tops = []
        for _ in range(_TOPK):
            mv = jnp.max(iv, axis=1, keepdims=True)
            sl = jnp.min(jnp.where(iv == mv, slot, big), axis=1, keepdims=True)
            iv = jnp.where(slot == sl, -1.0, iv)
            tops.append(mv)
        dynk = jnp.clip(jnp.floor(_sum10_like_xla(tops)), 1.0, float(_TOPK))

        sel_i, sel_v = [], []
        for _ in range(_TOPK):
            mv = jnp.min(av, axis=1, keepdims=True)
            an = jnp.min(jnp.where(av == mv, ai, big), axis=1, keepdims=True)
            av = jnp.where((av == mv) & (ai == an), inf, av)
            sel_i.append(an)
            sel_v.append(mv)

        iota_a = _fiota((G, A), 1)
        m = jnp.zeros((G, A), jnp.float32) > 0.0
        csel = jnp.full((G, A), inf, jnp.float32)
        for j in range(_TOPK):
            pickc = (dynk > float(j)) & (mg > 0.0)
            hit = (iota_a == sel_i[j]) & pickc
            m = m | hit
            csel = jnp.where(hit, sel_v[j], csel)

        m_f = jnp.where(m, 1.0, 0.0)
        cnt = jnp.sum(m_f, axis=0, keepdims=True)
        bv = jnp.min(csel, axis=0, keepdims=True)
        iota_g = _fiota((G, A), 0)
        bestg = jnp.min(jnp.where(csel == bv, iota_g, big), axis=0, keepdims=True)
        res_f = jnp.where((iota_g == bestg) & (cnt > 0.0), 1.0, 0.0)
        out_ref[0] = jnp.where(cnt > 1.0, res_f, m_f)


def kernel(pred_boxes, gt_boxes, mask_gt, pred_scores, gt_labels):
    BS, A, _ = pred_boxes.shape
    G = gt_boxes.shape[1]
    nt = A // _AT

    w1 = pred_boxes[..., 2] - pred_boxes[..., 0]
    h1 = pred_boxes[..., 3] - pred_boxes[..., 1] + _EPS
    t1 = jnp.arctan(w1 / h1)
    w2 = gt_boxes[..., 2] - gt_boxes[..., 0]
    h2 = gt_boxes[..., 3] - gt_boxes[..., 1] + _EPS
    t2 = jnp.arctan(w2 / h2)

    pbt = jnp.concatenate([jnp.transpose(pred_boxes, (0, 2, 1)), t1[:, None, :]], axis=1)
    pbt4 = jnp.transpose(pbt.reshape(BS, 5, nt, _AT), (0, 2, 1, 3))
    gpack = jnp.stack(
        [gt_boxes[..., 0], gt_boxes[..., 1], gt_boxes[..., 2], gt_boxes[..., 3],
         t2, mask_gt.astype(jnp.float32), gt_labels.astype(jnp.float32),
         jnp.zeros((BS, G), jnp.float32)], axis=-1)

    out = pl.pallas_call(
        _body,
        grid=(BS, nt),
        in_specs=[
            pl.BlockSpec((1, 1, 5, _AT), lambda b, t: (b, t, 0, 0)),
            pl.BlockSpec((1, G, 8), lambda b, t: (b, 0, 0)),
            pl.BlockSpec((1, _AT, _NC), lambda b, t: (b, t, 0)),
        ],
        out_specs=pl.BlockSpec((1, G, A), lambda b, t: (b, 0, 0)),
        out_shape=jax.ShapeDtypeStruct((BS, G, A), jnp.float32),
        scratch_shapes=[
            pltpu.VMEM((nt, G, _NSLOT), jnp.float32),
            pltpu.VMEM((nt, G, _NSLOT), jnp.float32),
            pltpu.VMEM((nt, G, _NSLOT), jnp.float32),
        ],
        compiler_params=pltpu.CompilerParams(
            dimension_semantics=("arbitrary", "arbitrary"),
        ),
    )(pbt4, gpack, pred_scores)
    return out


# speedup vs baseline: 13.4517x; 1.0927x over previous
import math

import jax
import jax.numpy as jnp
from jax import lax
from jax.experimental import pallas as pl
from jax.experimental.pallas import tpu as pltpu

_NC = 80
_TOPK = 10
_EPS = 1e-09
_DIS = 2.5
_AT = 2000
_NSLOT = 16


def _fiota(shape, dim):
    return lax.broadcasted_iota(jnp.int32, shape, dim).astype(jnp.float32)


def _sum_classes_like_xla(e):
    z = e[:, 0:8]
    for j in range(1, _NC // 8):
        z = z + e[:, 8 * j : 8 * j + 8]
    a = z[:, 0:4] + z[:, 4:8]
    b = a[:, 0:2] + a[:, 2:4]
    return b[:, 0:1] + b[:, 1:2]


def _sum10_like_xla(vals):
    z = [vals[i] + vals[8 + i] if 8 + i < len(vals) else vals[i] for i in range(8)]
    a = [z[i] + z[i + 4] for i in range(4)]
    b = [a[0] + a[2], a[1] + a[3]]
    return b[0] + b[1]


def _pack_slots(cols, pad):
    G = cols[0].shape[0]
    slot = _fiota((G, _NSLOT), 1)
    acc = jnp.full((G, _NSLOT), pad, jnp.float32)
    for j, c in enumerate(cols):
        acc = jnp.where(slot == float(j), c, acc)
    return acc


def _body(pbt_ref, g_ref, ps_ref, out_ref, cval_ref, cidx_ref, ival_ref):
    t = pl.program_id(1)
    nt = pl.num_programs(1)
    G = g_ref.shape[1]
    A = out_ref.shape[2]

    x1 = pbt_ref[0, 0, 0:1, :]
    y1 = pbt_ref[0, 0, 1:2, :]
    x2 = pbt_ref[0, 0, 2:3, :]
    y2 = pbt_ref[0, 0, 3:4, :]
    t1 = pbt_ref[0, 0, 4:5, :]
    gx1 = g_ref[0, :, 0:1]
    gy1 = g_ref[0, :, 1:2]
    gx2 = g_ref[0, :, 2:3]
    gy2 = g_ref[0, :, 3:4]
    t2 = g_ref[0, :, 4:5]
    mg = g_ref[0, :, 5:6]
    glf = g_ref[0, :, 6:7]

    cx = (x1 + x2) / 2.0
    cy = (y1 + y2) / 2.0
    in_boxes = (cx - gx1 > 0.0) & (cy - gy1 > 0.0) & (gx2 - cx > 0.0) & (gy2 - cy > 0.0)
    gcx = (gx1 + gx2) / 2.0
    gcy = (gy1 + gy2) / 2.0
    b2x1 = gcx - _DIS
    b2y1 = gcy - _DIS
    b2x2 = gcx + _DIS
    b2y2 = gcy + _DIS
    in_centers = (cx - b2x1 > 0.0) & (cy - b2y1 > 0.0) & (b2x2 - cx > 0.0) & (b2y2 - cy > 0.0)
    both = in_boxes & in_centers

    w1 = x2 - x1
    h1 = y2 - y1 + _EPS
    w2 = gx2 - gx1
    h2 = gy2 - gy1 + _EPS
    iw = jnp.maximum(jnp.minimum(x2, gx2) - jnp.maximum(x1, gx1), 0.0)
    ih = jnp.maximum(jnp.minimum(y2, gy2) - jnp.maximum(y1, gy1), 0.0)
    inter = iw * ih
    union = w1 * h1 + w2 * h2 - inter + _EPS
    iou = inter / union
    cw = jnp.maximum(x2, gx2) - jnp.minimum(x1, gx1)
    ch = jnp.maximum(y2, gy2) - jnp.minimum(y1, gy1)
    c2 = cw * cw + ch * ch + _EPS
    dx = gx1 + gx2 - x1 - x2
    dy = gy1 + gy2 - y1 - y2
    d2 = (dx * dx + dy * dy) / 4.0
    tdiff = t2 - t1
    v = 4.0 / math.pi**2 * (tdiff * tdiff)
    alpha = v / (v - iou + (1.0 + _EPS))
    ciou = iou - (d2 / c2 + v * alpha)

    p = jnp.clip(ps_ref[0], 1e-07, 1.0 - 1e-07)
    log1mp = jnp.log1p(-p)
    s_col = _sum_classes_like_xla(-log1mp)
    onehot = jnp.where(_fiota((G, _NC), 1) == glf, 1.0, 0.0)
    dims = (((1,), (1,)), ((), ()))
    hi = jax.lax.Precision.HIGHEST
    p_g = lax.dot_general(onehot, p, dims, precision=hi, preferred_element_type=jnp.float32)
    lp_g = jnp.log(p_g)
    l1_g = lax.dot_general(onehot, log1mp, dims, precision=hi, preferred_element_type=jnp.float32)
    ones_g = jnp.ones((G, 1), jnp.float32)
    s_row = lax.dot_general(ones_g, s_col, dims, precision=hi, preferred_element_type=jnp.float32)
    bce = (s_row - lp_g + l1_g) / float(_NC)

    cost = bce + 3.0 * ciou + 100000.0 * (1.0 - jnp.where(both, 1.0, 0.0))
    cost = cost + 1e9 * (1.0 - mg)
    iou_pos = jnp.clip(iou, 0.0, 1.0)

    lane = _fiota((G, _AT), 1)
    big = jnp.float32(1e9)
    inf = jnp.float32(jnp.inf)
    base = (t * _AT).astype(jnp.float32)

    cvals, cidxs = [], []
    work = cost
    for _ in range(_TOPK):
        mv = jnp.min(work, axis=1, keepdims=True)
        li = jnp.min(jnp.where(work == mv, lane, big), axis=1, keepdims=True)
        work = jnp.where(lane == li, inf, work)
        cvals.append(mv)
        cidxs.append(li + base)
    cval_ref[t] = _pack_slots(cvals, jnp.inf)
    cidx_ref[t] = _pack_slots(cidxs, 1e9)

    ivals = []
    work = iou_pos
    for _ in range(_TOPK):
        mv = jnp.max(work, axis=1, keepdims=True)
        li = jnp.min(jnp.where(work == mv, lane, big), axis=1, keepdims=True)
        work = jnp.where(lane == li, -1.0, work)
        ivals.append(mv)
    ival_ref[t] = _pack_slots(ivals, -1.0)

    @pl.when(t == nt - 1)
    def _finalize():
        av = jnp.concatenate([cval_ref[tt] for tt in range(nt)], axis=1)
        ai = jnp.concatenate([cidx_ref[tt] for tt in range(nt)], axis=1)
        iv = jnp.concatenate([ival_ref[tt] for tt in range(nt)], axis=1)
        nslots = nt * _NSLOT
        slot = _fiota((G, nslots), 1)

        tops = []
        for _ in range(_TOPK):
            mv = jnp.max(iv, axis=1, keepdims=True)
            sl = jnp.min(jnp.where(iv == mv, slot, big), axis=1, keepdims=True)
            iv = jnp.where(slot == sl, -1.0, iv)
            tops.append(mv)
        dynk = jnp.clip(jnp.floor(_sum10_like_xla(tops)), 1.0, float(_TOPK))

        sel_i, sel_v = [], []
        for _ in range(_TOPK):
            mv = jnp.min(av, axis=1, keepdims=True)
            an = jnp.min(jnp.where(av == mv, ai, big), axis=1, keepdims=True)
            av = jnp.where((av == mv) & (ai == an), inf, av)
            sel_i.append(an)
            sel_v.append(mv)

        iota_a = _fiota((G, A), 1)
        csel = jnp.full((G, A), inf, jnp.float32)
        for j in range(_TOPK):
            pickc = (dynk > float(j)) & (mg > 0.0)
            smask = jnp.where(pickc, sel_i[j], -1.0)
            csel = jnp.where(iota_a == smask, sel_v[j], csel)

        bv = jnp.min(csel, axis=0, keepdims=True)
        iota_g = _fiota((G, A), 0)
        bestg = jnp.min(jnp.where(csel == bv, iota_g, big), axis=0, keepdims=True)
        out_ref[0] = jnp.where((iota_g == bestg) & (bv < inf), 1.0, 0.0)


def kernel(pred_boxes, gt_boxes, mask_gt, pred_scores, gt_labels):
    BS, A, _ = pred_boxes.shape
    G = gt_boxes.shape[1]
    nt = A // _AT

    w1 = pred_boxes[..., 2] - pred_boxes[..., 0]
    h1 = pred_boxes[..., 3] - pred_boxes[..., 1] + _EPS
    t1 = jnp.arctan(w1 / h1)
    w2 = gt_boxes[..., 2] - gt_boxes[..., 0]
    h2 = gt_boxes[..., 3] - gt_boxes[..., 1] + _EPS
    t2 = jnp.arctan(w2 / h2)

    pbt = jnp.concatenate([jnp.transpose(pred_boxes, (0, 2, 1)), t1[:, None, :]], axis=1)
    pbt4 = jnp.transpose(pbt.reshape(BS, 5, nt, _AT), (0, 2, 1, 3))
    gpack = jnp.stack(
        [gt_boxes[..., 0], gt_boxes[..., 1], gt_boxes[..., 2], gt_boxes[..., 3],
         t2, mask_gt.astype(jnp.float32), gt_labels.astype(jnp.float32),
         jnp.zeros((BS, G), jnp.float32)], axis=-1)

    out = pl.pallas_call(
        _body,
        grid=(BS, nt),
        in_specs=[
            pl.BlockSpec((1, 1, 5, _AT), lambda b, t: (b, t, 0, 0)),
            pl.BlockSpec((1, G, 8), lambda b, t: (b, 0, 0)),
            pl.BlockSpec((1, _AT, _NC), lambda b, t: (b, t, 0)),
        ],
        out_specs=pl.BlockSpec((1, G, A), lambda b, t: (b, 0, 0)),
        out_shape=jax.ShapeDtypeStruct((BS, G, A), jnp.float32),
        scratch_shapes=[
            pltpu.VMEM((nt, G, _NSLOT), jnp.float32),
            pltpu.VMEM((nt, G, _NSLOT), jnp.float32),
            pltpu.VMEM((nt, G, _NSLOT), jnp.float32),
        ],
        compiler_params=pltpu.CompilerParams(
            dimension_semantics=("arbitrary", "arbitrary"),
        ),
    )(pbt4, gpack, pred_scores)
    return out


# speedup vs baseline: 14.4824x; 1.0766x over previous
import math

import jax
import jax.numpy as jnp
from jax import lax
from jax.experimental import pallas as pl
from jax.experimental.pallas import tpu as pltpu

_NC = 80
_TOPK = 10
_EPS = 1e-09
_DIS = 2.5
_AT = 5000
_GB = 32
_NSLOT = 16


def _fiota(shape, dim):
    return lax.broadcasted_iota(jnp.int32, shape, dim).astype(jnp.float32)


def _sum_classes_like_xla(e):
    z = e[:, 0:8]
    for j in range(1, _NC // 8):
        z = z + e[:, 8 * j : 8 * j + 8]
    a = z[:, 0:4] + z[:, 4:8]
    b = a[:, 0:2] + a[:, 2:4]
    return b[:, 0:1] + b[:, 1:2]


def _sum10_like_xla(vals):
    z = [vals[i] + vals[8 + i] if 8 + i < len(vals) else vals[i] for i in range(8)]
    a = [z[i] + z[i + 4] for i in range(4)]
    b = [a[0] + a[2], a[1] + a[3]]
    return b[0] + b[1]


def _pack_slots(cols, pad):
    G = cols[0].shape[0]
    slot = _fiota((G, _NSLOT), 1)
    acc = jnp.full((G, _NSLOT), pad, jnp.float32)
    for j, c in enumerate(cols):
        acc = jnp.where(slot == float(j), c, acc)
    return acc


def _body(pbt_ref, g_ref, ps_ref, out_ref, cval_ref, cidx_ref, ival_ref):
    t = pl.program_id(1)
    nt = pl.num_programs(1)
    G = g_ref.shape[1]
    A = out_ref.shape[2]

    x1 = pbt_ref[0, 0, 0:1, :]
    y1 = pbt_ref[0, 0, 1:2, :]
    x2 = pbt_ref[0, 0, 2:3, :]
    y2 = pbt_ref[0, 0, 3:4, :]
    t1 = pbt_ref[0, 0, 4:5, :]
    p = jnp.clip(ps_ref[0], 1e-07, 1.0 - 1e-07)
    log1mp = jnp.log1p(-p)
    s_col = _sum_classes_like_xla(-log1mp)
    dims = (((1,), (1,)), ((), ()))
    hi = jax.lax.Precision.HIGHEST

    lane = _fiota((_GB, _AT), 1)
    big = jnp.float32(1e9)
    inf = jnp.float32(jnp.inf)
    base = (t * _AT).astype(jnp.float32)

    for gb in range(g_ref.shape[1] // _GB):
        sl = slice(gb * _GB, (gb + 1) * _GB)
        gx1 = g_ref[0, sl, 0:1]
        gy1 = g_ref[0, sl, 1:2]
        gx2 = g_ref[0, sl, 2:3]
        gy2 = g_ref[0, sl, 3:4]
        t2 = g_ref[0, sl, 4:5]
        mg = g_ref[0, sl, 5:6]
        glf = g_ref[0, sl, 6:7]

        cx = (x1 + x2) / 2.0
        cy = (y1 + y2) / 2.0
        in_boxes = (cx - gx1 > 0.0) & (cy - gy1 > 0.0) & (gx2 - cx > 0.0) & (gy2 - cy > 0.0)
        gcx = (gx1 + gx2) / 2.0
        gcy = (gy1 + gy2) / 2.0
        b2x1 = gcx - _DIS
        b2y1 = gcy - _DIS
        b2x2 = gcx + _DIS
        b2y2 = gcy + _DIS
        in_centers = (cx - b2x1 > 0.0) & (cy - b2y1 > 0.0) & (b2x2 - cx > 0.0) & (b2y2 - cy > 0.0)
        both = in_boxes & in_centers

        w1 = x2 - x1
        h1 = y2 - y1 + _EPS
        w2 = gx2 - gx1
        h2 = gy2 - gy1 + _EPS
        iw = jnp.maximum(jnp.minimum(x2, gx2) - jnp.maximum(x1, gx1), 0.0)
        ih = jnp.maximum(jnp.minimum(y2, gy2) - jnp.maximum(y1, gy1), 0.0)
        inter = iw * ih
        union = w1 * h1 + w2 * h2 - inter + _EPS
        iou = inter / union
        cw = jnp.maximum(x2, gx2) - jnp.minimum(x1, gx1)
        ch = jnp.maximum(y2, gy2) - jnp.minimum(y1, gy1)
        c2 = cw * cw + ch * ch + _EPS
        dx = gx1 + gx2 - x1 - x2
        dy = gy1 + gy2 - y1 - y2
        d2 = (dx * dx + dy * dy) / 4.0
        tdiff = t2 - t1
        v = 4.0 / math.pi**2 * (tdiff * tdiff)
        alpha = v / (v - iou + (1.0 + _EPS))
        ciou = iou - (d2 / c2 + v * alpha)

        onehot = jnp.where(_fiota((_GB, _NC), 1) == glf, 1.0, 0.0)
        p_g = lax.dot_general(onehot, p, dims, precision=hi, preferred_element_type=jnp.float32)
        lp_g = jnp.log(p_g)
        l1_g = lax.dot_general(onehot, log1mp, dims, precision=hi, preferred_element_type=jnp.float32)
        ones_g = jnp.ones((_GB, 1), jnp.float32)
        s_row = lax.dot_general(ones_g, s_col, dims, precision=hi, preferred_element_type=jnp.float32)
        bce = (s_row - lp_g + l1_g) / float(_NC)

        cost = bce + 3.0 * ciou + 100000.0 * (1.0 - jnp.where(both, 1.0, 0.0))
        cost = cost + 1e9 * (1.0 - mg)
        iou_pos = jnp.clip(iou, 0.0, 1.0)

        cvals, cidxs, ivals = [], [], []
        workc = cost
        worki = iou_pos
        for _ in range(_TOPK):
            mvc = jnp.min(workc, axis=1, keepdims=True)
            mvi = jnp.max(worki, axis=1, keepdims=True)
            lic = jnp.min(jnp.where(workc == mvc, lane, big), axis=1, keepdims=True)
            lii = jnp.min(jnp.where(worki == mvi, lane, big), axis=1, keepdims=True)
            workc = jnp.where(lane == lic, inf, workc)
            worki = jnp.where(lane == lii, -1.0, worki)
            cvals.append(mvc)
            cidxs.append(lic + base)
            ivals.append(mvi)
        cval_ref[t, sl, :] = _pack_slots(cvals, jnp.inf)
        cidx_ref[t, sl, :] = _pack_slots(cidxs, 1e9)
        ival_ref[t, sl, :] = _pack_slots(ivals, -1.0)

    @pl.when(t == nt - 1)
    def _finalize():
        mg = g_ref[0, :, 5:6]
        av = jnp.concatenate([cval_ref[tt] for tt in range(nt)], axis=1)
        ai = jnp.concatenate([cidx_ref[tt] for tt in range(nt)], axis=1)
        iv = jnp.concatenate([ival_ref[tt] for tt in range(nt)], axis=1)
        nslots = nt * _NSLOT
        slot = _fiota((G, nslots), 1)

        tops = []
        for _ in range(_TOPK):
            mv = jnp.max(iv, axis=1, keepdims=True)
            sl = jnp.min(jnp.where(iv == mv, slot, big), axis=1, keepdims=True)
            iv = jnp.where(slot == sl, -1.0, iv)
            tops.append(mv)
        dynk = jnp.clip(jnp.floor(_sum10_like_xla(tops)), 1.0, float(_TOPK))

        sel_i, sel_v = [], []
        for _ in range(_TOPK):
            mv = jnp.min(av, axis=1, keepdims=True)
            an = jnp.min(jnp.where(av == mv, ai, big), axis=1, keepdims=True)
            av = jnp.where((av == mv) & (ai == an), inf, av)
            sel_i.append(an)
            sel_v.append(mv)

        iota_a = _fiota((G, A), 1)
        csel = jnp.full((G, A), inf, jnp.float32)
        for j in range(_TOPK):
            pickc = (dynk > float(j)) & (mg > 0.0)
            smask = jnp.where(pickc, sel_i[j], -1.0)
            csel = jnp.where(iota_a == smask, sel_v[j], csel)

        bv = jnp.min(csel, axis=0, keepdims=True)
        iota_g = _fiota((G, A), 0)
        bestg = jnp.min(jnp.where(csel == bv, iota_g, big), axis=0, keepdims=True)
        out_ref[0] = jnp.where((iota_g == bestg) & (bv < inf), 1.0, 0.0)


def kernel(pred_boxes, gt_boxes, mask_gt, pred_scores, gt_labels):
    BS, A, _ = pred_boxes.shape
    G = gt_boxes.shape[1]
    nt = A // _AT

    w1 = pred_boxes[..., 2] - pred_boxes[..., 0]
    h1 = pred_boxes[..., 3] - pred_boxes[..., 1] + _EPS
    t1 = jnp.arctan(w1 / h1)
    w2 = gt_boxes[..., 2] - gt_boxes[..., 0]
    h2 = gt_boxes[..., 3] - gt_boxes[..., 1] + _EPS
    t2 = jnp.arctan(w2 / h2)

    pbt = jnp.concatenate([jnp.transpose(pred_boxes, (0, 2, 1)), t1[:, None, :]], axis=1)
    pbt4 = jnp.transpose(pbt.reshape(BS, 5, nt, _AT), (0, 2, 1, 3))
    gpack = jnp.stack(
        [gt_boxes[..., 0], gt_boxes[..., 1], gt_boxes[..., 2], gt_boxes[..., 3],
         t2, mask_gt.astype(jnp.float32), gt_labels.astype(jnp.float32),
         jnp.zeros((BS, G), jnp.float32)], axis=-1)

    out = pl.pallas_call(
        _body,
        grid=(BS, nt),
        in_specs=[
            pl.BlockSpec((1, 1, 5, _AT), lambda b, t: (b, t, 0, 0)),
            pl.BlockSpec((1, G, 8), lambda b, t: (b, 0, 0)),
            pl.BlockSpec((1, _AT, _NC), lambda b, t: (b, t, 0)),
        ],
        out_specs=pl.BlockSpec((1, G, A), lambda b, t: (b, 0, 0)),
        out_shape=jax.ShapeDtypeStruct((BS, G, A), jnp.float32),
        scratch_shapes=[
            pltpu.VMEM((nt, G, _NSLOT), jnp.float32),
            pltpu.VMEM((nt, G, _NSLOT), jnp.float32),
            pltpu.VMEM((nt, G, _NSLOT), jnp.float32),
        ],
        compiler_params=pltpu.CompilerParams(
            dimension_semantics=("arbitrary", "arbitrary"),
        ),
    )(pbt4, gpack, pred_scores)
    return out
